# initial kernel scaffold (unmeasured)
import jax
import jax.numpy as jnp
from jax import lax
from jax.experimental import pallas as pl
from jax.experimental.pallas import tpu as pltpu

N_DEV = 32


def kernel(x, w_mat, scale_x, scale_w):
    M, k_per = x.shape
    K, N = w_mat.shape
    m_per = M // N_DEV

    def body(x_ref, w_ref, sx_ref, sw_ref, out_ref, xt_ref, send_sems, recv_sems):
        my = lax.axis_index("i")

        xt_ref[:, pl.ds(my * k_per, k_per)] = x_ref[pl.ds(my * m_per, m_per), :]

        sends = []
        for s in range(1, N_DEV):
            dst = lax.rem(my + s, N_DEV)
            rdma = pltpu.make_async_remote_copy(
                src_ref=x_ref.at[pl.ds(dst * m_per, m_per), :],
                dst_ref=xt_ref.at[:, pl.ds(my * k_per, k_per)],
                send_sem=send_sems.at[s],
                recv_sem=recv_sems.at[my],
                device_id=(dst,),
                device_id_type=pl.DeviceIdType.MESH,
            )
            rdma.start()
            sends.append(rdma)

        for s in range(1, N_DEV):
            src = lax.rem(my + s, N_DEV)
            recv = pltpu.make_async_remote_copy(
                src_ref=x_ref.at[pl.ds(0, m_per), :],
                dst_ref=xt_ref.at[:, pl.ds(src * k_per, k_per)],
                send_sem=send_sems.at[s],
                recv_sem=recv_sems.at[src],
                device_id=(src,),
                device_id_type=pl.DeviceIdType.MESH,
            )
            recv.wait_recv()

        acc = lax.dot_general(
            xt_ref[:, :], w_ref[:, :],
            (((1,), (0,)), ((), ())),
            preferred_element_type=jnp.float32,
        )
        y = acc * (sx_ref[0] * sw_ref[0])
        out_ref[:, :] = y * jax.nn.sigmoid(y)

        for rdma in sends:
            rdma.wait_send()

    return pl.pallas_call(
        body,
        out_shape=jax.ShapeDtypeStruct((m_per, N), jnp.float32),
        in_specs=[
            pl.BlockSpec(memory_space=pltpu.VMEM),
            pl.BlockSpec(memory_space=pltpu.VMEM),
            pl.BlockSpec(memory_space=pltpu.SMEM),
            pl.BlockSpec(memory_space=pltpu.SMEM),
        ],
        out_specs=pl.BlockSpec(memory_space=pltpu.VMEM),
        scratch_shapes=[
            pltpu.VMEM((m_per, K), x.dtype),
            pltpu.SemaphoreType.DMA((N_DEV,)),
            pltpu.SemaphoreType.DMA((N_DEV,)),
        ],
        compiler_params=pltpu.CompilerParams(
            vmem_limit_bytes=100 * 1024 * 1024,
        ),
    )(x, w_mat, scale_x, scale_w)


# baseline (device time: 63649 ns/iter reference)
import jax
import jax.numpy as jnp
from jax import lax
from jax.experimental import pallas as pl
from jax.experimental.pallas import tpu as pltpu

N_DEV = 32
BN = 512

F8 = jnp.float8_e4m3fn


def kernel(x, w_mat, scale_x, scale_w):
    M, k_per = x.shape
    K, N = w_mat.shape
    m_per = M // N_DEV
    n_steps = N // BN

    def body(x_ref, w_ref, sx_ref, sw_ref, out_ref, x8_ref, xt_ref,
             send_sems, recv_sems):
        j = pl.program_id(0)
        my = lax.axis_index("i")

        @pl.when(j == 0)
        def _comm():
            barrier_sem = pltpu.get_barrier_semaphore()
            for s in range(1, N_DEV):
                peer = lax.rem(my + s, N_DEV)
                pl.semaphore_signal(
                    barrier_sem, inc=1,
                    device_id=(peer,), device_id_type=pl.DeviceIdType.MESH,
                )
            pl.semaphore_wait(barrier_sem, N_DEV - 1)

            x8_ref[:, :] = x_ref[:, :].astype(F8)
            xt_ref[:, pl.ds(my * k_per, k_per)] = x8_ref[pl.ds(my * m_per, m_per), :]

            sends = []
            for s in range(1, N_DEV):
                dst = lax.rem(my + s, N_DEV)
                rdma = pltpu.make_async_remote_copy(
                    src_ref=x8_ref.at[pl.ds(dst * m_per, m_per), :],
                    dst_ref=xt_ref.at[:, pl.ds(my * k_per, k_per)],
                    send_sem=send_sems.at[s],
                    recv_sem=recv_sems.at[my],
                    device_id=(dst,),
                    device_id_type=pl.DeviceIdType.MESH,
                )
                rdma.start()
                sends.append(rdma)

            for s in range(1, N_DEV):
                src = lax.rem(my + s, N_DEV)
                recv = pltpu.make_async_remote_copy(
                    src_ref=x8_ref.at[pl.ds(0, m_per), :],
                    dst_ref=xt_ref.at[:, pl.ds(src * k_per, k_per)],
                    send_sem=send_sems.at[s],
                    recv_sem=recv_sems.at[src],
                    device_id=(src,),
                    device_id_type=pl.DeviceIdType.MESH,
                )
                recv.wait_recv()

            for rdma in sends:
                rdma.wait_send()

        w8 = w_ref[:, :].astype(F8)
        acc = lax.dot_general(
            xt_ref[:, :], w8,
            (((1,), (0,)), ((), ())),
            preferred_element_type=jnp.float32,
        )
        y = acc * (sx_ref[0] * sw_ref[0])
        out_ref[:, :] = y * jax.nn.sigmoid(y)

    return pl.pallas_call(
        body,
        grid=(n_steps,),
        out_shape=jax.ShapeDtypeStruct((m_per, N), jnp.float32),
        in_specs=[
            pl.BlockSpec((M, k_per), lambda j: (0, 0)),
            pl.BlockSpec((K, BN), lambda j: (0, j)),
            pl.BlockSpec(memory_space=pltpu.SMEM),
            pl.BlockSpec(memory_space=pltpu.SMEM),
        ],
        out_specs=pl.BlockSpec((m_per, BN), lambda j: (0, j)),
        scratch_shapes=[
            pltpu.VMEM((M, k_per), F8),
            pltpu.VMEM((m_per, K), F8),
            pltpu.SemaphoreType.DMA((N_DEV,)),
            pltpu.SemaphoreType.DMA((N_DEV,)),
        ],
        compiler_params=pltpu.CompilerParams(
            collective_id=0,
            vmem_limit_bytes=60 * 1024 * 1024,
        ),
    )(x, w_mat, scale_x, scale_w)
